# Initial kernel scaffold; baseline (speedup 1.0000x reference)
import jax
import jax.numpy as jnp
from jax.experimental import pallas as pl


def kernel(keypoints):
    raise NotImplementedError("write your pallas kernel here")


# speedup vs baseline: 37.8021x; 37.8021x over previous
import functools

import jax
import jax.numpy as jnp
from jax.experimental import pallas as pl
from jax.experimental.pallas import tpu as pltpu

NUM_CLIPS = 10
CLIP_LEN = 100
THRESHOLD = 0.01
W = 960.0
H = 576.0
NUM_PERSON = 2
V = 17
C = 3
D = V * C


def _tc_body(starts_ref, kp_ref, out_ref, scratch):
    T = kp_ref.shape[1]
    scratch[0:T, :] = kp_ref[0]
    scratch[T:T + CLIP_LEN, :] = kp_ref[0, 0:CLIP_LEN, :]
    b = pl.program_id(0)
    lane = jax.lax.broadcasted_iota(jnp.int32, (CLIP_LEN, D), 1)
    ch = lane % 3
    scale = jnp.where(ch == 0, 2.0 / W,
                      jnp.where(ch == 1, 2.0 / H, 1.0)).astype(jnp.float32)
    offset = jnp.where(ch == 2, 0.0, -1.0).astype(jnp.float32)
    for c in range(NUM_CLIPS):
        start = starts_ref[b * NUM_CLIPS + c]
        v = scratch[pl.ds(start, CLIP_LEN), :]
        conf = jnp.where(ch == 0, jnp.roll(v, -2, axis=1),
                         jnp.where(ch == 1, jnp.roll(v, -1, axis=1), v))
        nv = v * scale + offset
        out_ref[0, c] = jnp.where(ch == 2, v,
                                  jnp.where(conf <= THRESHOLD, 0.0, nv))


def _clip_starts(B, T):
    keys = jax.random.split(jax.random.key(42), B)
    starts = jax.vmap(
        lambda k: jax.random.randint(k, (NUM_CLIPS,), 0, T))(keys)
    return starts.reshape(-1).astype(jnp.int32)


def kernel(keypoints):
    B, T = keypoints.shape[0], keypoints.shape[1]
    kp = keypoints.reshape(B, T, D)
    starts = _clip_starts(B, T)
    out_c = pl.pallas_call(
        _tc_body,
        grid_spec=pltpu.PrefetchScalarGridSpec(
            num_scalar_prefetch=1,
            grid=(B,),
            in_specs=[pl.BlockSpec((1, T, D), lambda b, s: (b, 0, 0))],
            out_specs=pl.BlockSpec((1, NUM_CLIPS, CLIP_LEN, D),
                                   lambda b, s: (b, 0, 0, 0)),
            scratch_shapes=[pltpu.VMEM((T + CLIP_LEN, D), jnp.float32)],
        ),
        out_shape=jax.ShapeDtypeStruct((B, NUM_CLIPS, CLIP_LEN, D),
                                       jnp.float32),
    )(starts, kp)
    out = out_c.reshape(B, NUM_CLIPS, 1, CLIP_LEN, V, C)
    zeros = jnp.zeros_like(out)
    return jnp.concatenate([out, zeros], axis=2)
